# baseline (speedup 1.0000x reference)
import functools

import jax
import jax.numpy as jnp
import numpy as np
from jax import lax
from jax.experimental import pallas as pl
from jax.experimental.pallas import tpu as pltpu
from jax.experimental.pallas import tpu_sc as plsc

N_NODES = 10000
NPAD = 10240
D = 128
DH = 64
E_EDGES = 320000
EROWS_REAL = E_EDGES // 128
EROWS_DEG = 2560
EROWS = 2688
EPAD = EROWS * 128
NC = 2
NS = 16
KDEG = EROWS_DEG // (NC * NS)
EB = 192
EBROWS = EROWS * 128 // EB
KAGG = EBROWS // NS
RPT = NPAD // NS
BLK = 1024
GRID = NPAD // BLK

NBUF = 4
NRING = 6
STAG = 2


def _col_perm():
    perm = np.empty((D,), np.int64)
    for b0 in range(0, D, 32):
        perm[b0:b0 + 32:2] = b0 + np.arange(16)
        perm[b0 + 1:b0 + 32:2] = b0 + 16 + np.arange(16)
    return perm


def _sc_mesh():
    return plsc.VectorSubcoreMesh(core_axis_name="c", subcore_axis_name="s",
                                  num_cores=NC, num_subcores=NS)


def _deg_body(dstx_hbm, out_hbm, dst_v, ones_v, z_v, acc):
    cid = lax.axis_index("c")
    sid = lax.axis_index("s")
    wid = cid * NS + sid

    pltpu.sync_copy(dstx_hbm.at[pl.ds(wid * KDEG, KDEG)], dst_v)

    def fill(i, _):
        ones_v[pl.ds(i * 16, 16)] = jnp.full((16,), 1.0, jnp.float32)
        z_v[pl.ds(i * 16, 16)] = jnp.zeros((16,), jnp.float32)
        return 0

    lax.fori_loop(0, 8, fill, 0)

    def zcp(t, _):
        pltpu.sync_copy(z_v, acc.at[pl.ds(sid * RPT + t * 128, 128)])
        return 0

    lax.fori_loop(0, RPT // 128, zcp, 0)
    plsc.subcore_barrier()

    def body(j, _):
        pltpu.sync_copy(ones_v, acc.at[dst_v.at[j]], add=True)
        return 0

    lax.fori_loop(0, KDEG, body, 0)
    plsc.subcore_barrier()
    pltpu.sync_copy(
        acc.at[pl.ds(sid * RPT, RPT)],
        out_hbm.at[cid, pl.ds(sid * RPT, RPT)],
    )


@functools.cache
def _build_deg_kernel():
    return functools.partial(
        pl.kernel,
        out_type=jax.ShapeDtypeStruct((NC, NPAD), jnp.float32),
        mesh=_sc_mesh(),
        scratch_types=[
            pltpu.VMEM((KDEG, 128), jnp.int32),
            pltpu.VMEM((128,), jnp.float32),
            pltpu.VMEM((128,), jnp.float32),
            pltpu.VMEM_SHARED((NPAD,), jnp.float32),
        ],
    )(_deg_body)


def _deg_kernel(dstx):
    return _build_deg_kernel()(dstx)


def _agg_half(xs_hbm, edgx_hbm, out_hbm, sid, ribuf, gbufs, fbufs, isems,
              gsems, ssems, z_v, acc):
    base = sid * KAGG

    def zcp(t, _):
        pltpu.sync_copy(z_v, acc.at[pl.ds(sid * RPT + t * 64, 64)])
        return 0

    lax.fori_loop(0, RPT // 64, zcp, 0)
    plsc.subcore_barrier()

    def i_desc(j):
        return pltpu.make_async_copy(edgx_hbm.at[base + j],
                                     ribuf.at[lax.rem(j, NRING)],
                                     isems[0])

    def g_desc(j, b):
        return pltpu.make_async_copy(
            xs_hbm.at[ribuf.at[lax.rem(j, NRING), 0]], gbufs[b], gsems[b])

    def s_desc(j, b):
        return pltpu.make_async_copy(
            fbufs[b], acc.at[ribuf.at[lax.rem(j, NRING), 1]], ssems[b])

    def s_start(j, b):
        pltpu.async_copy(fbufs[b], acc.at[ribuf.at[lax.rem(j, NRING), 1]],
                         ssems[b], add=True)

    def convert(b):
        gbuf = gbufs[b]
        fbuf = fbufs[b]

        def conv_row(r, _):
            for g in range(2):
                ab = gbuf[r, pl.ds(g * 32, 32)]
                lo16, hi16 = plsc.unpack(
                    ab, format=plsc.PackFormat.INTERLEAVED,
                    preferred_element_type=jnp.float32)
                fbuf[r, pl.ds(g * 32, 16)] = lo16
                fbuf[r, pl.ds(g * 32 + 16, 16)] = hi16
            return 0

        lax.fori_loop(0, EB, conv_row, 0)

    for j0 in range(NRING):
        i_desc(j0).start()
    for b0 in range(STAG):
        i_desc(b0).wait()
        g_desc(b0, b0).start()

    def body(i, _):
        for b in range(NBUF):
            j = i * NBUF + b
            g_desc(j, b).wait()
            convert(b)
            s_start(j, b)
            j2 = j - STAG
            b2 = (b + STAG) % NBUF

            @pl.when(j2 + NBUF < KAGG)
            def _():
                i_desc(j2 + NBUF).wait()
                g_desc(j2 + NBUF, b2).start()

            @pl.when(j2 >= 0)
            def _():
                s_desc(j2, b2).wait()

            @pl.when(jnp.logical_and(j2 >= 0, j2 + NRING < KAGG))
            def _():
                i_desc(j2 + NRING).start()

        return 0

    lax.fori_loop(0, KAGG // NBUF, body, 0)
    for jt in range(KAGG - STAG, KAGG):
        s_desc(jt, jt % NBUF).wait()
    plsc.subcore_barrier()
    pltpu.sync_copy(
        acc.at[pl.ds(sid * RPT, RPT)],
        out_hbm.at[pl.ds(sid * RPT, RPT)],
    )


def _agg_body(xs_lo_hbm, xs_hi_hbm, edgx_hbm, out_lo_hbm,
              out_hi_hbm, ribuf, gbuf0, gbuf1, gbuf2, gbuf3,
              fbuf0, fbuf1, fbuf2, fbuf3, isem,
              gsem0, gsem1, gsem2, gsem3, ssem0, ssem1, ssem2, ssem3,
              z_v, acc):
    cid = lax.axis_index("c")
    sid = lax.axis_index("s")

    def zfill(i, _):
        for k in range(4):
            z_v[i, pl.ds(k * 16, 16)] = jnp.zeros((16,), jnp.float32)
        return 0

    lax.fori_loop(0, 64, zfill, 0)

    gbufs = (gbuf0, gbuf1, gbuf2, gbuf3)
    fbufs = (fbuf0, fbuf1, fbuf2, fbuf3)
    isems = (isem,)
    gsems = (gsem0, gsem1, gsem2, gsem3)
    ssems = (ssem0, ssem1, ssem2, ssem3)

    @pl.when(cid == 0)
    def _():
        _agg_half(xs_lo_hbm, edgx_hbm, out_lo_hbm, sid, ribuf, gbufs, fbufs,
                  isems, gsems, ssems, z_v, acc)

    @pl.when(cid == 1)
    def _():
        _agg_half(xs_hi_hbm, edgx_hbm, out_hi_hbm, sid, ribuf, gbufs, fbufs,
                  isems, gsems, ssems, z_v, acc)


@functools.cache
def _build_agg_kernel():
    return functools.partial(
        pl.kernel,
        out_type=[
            jax.ShapeDtypeStruct((NPAD, DH), jnp.float32),
            jax.ShapeDtypeStruct((NPAD, DH), jnp.float32),
        ],
        mesh=_sc_mesh(),
        scratch_types=(
            [pltpu.VMEM((NRING, 2, EB), jnp.int32)]
            + [pltpu.VMEM((EB, DH), jnp.bfloat16)] * NBUF
            + [pltpu.VMEM((EB, DH), jnp.float32)] * NBUF
            + [pltpu.SemaphoreType.DMA] * (1 + 2 * NBUF)
            + [
                pltpu.VMEM((64, DH), jnp.float32),
                pltpu.VMEM_SHARED((NPAD, DH), jnp.float32),
            ]
        ),
        compiler_params=pltpu.CompilerParams(use_tc_tiling_on_sc=False,
                                             needs_layout_passes=False),
    )(_agg_body)


def _agg_kernel(xs_lo, xs_hi, edgx):
    return _build_agg_kernel()(xs_lo, xs_hi, edgx)


def _row_iota(i):
    return lax.broadcasted_iota(jnp.int32, (BLK, 1), 0) + i * BLK


def _c1_body(degp_ref, x_ref, w_ref, dinv_ref, lo_ref, hi_ref):
    i = pl.program_id(0)
    deg = degp_ref[0] + degp_ref[1] + 1.0
    dinv = jnp.where(_row_iota(i) < N_NODES, lax.rsqrt(deg), 0.0)
    dinv_ref[...] = dinv
    xs = dinv * jnp.dot(x_ref[...], w_ref[...],
                        preferred_element_type=jnp.float32)
    lo_ref[...] = xs[:, :DH].astype(jnp.bfloat16)
    hi_ref[...] = xs[:, DH:].astype(jnp.bfloat16)


def _mid_body(plo_ref, phi_ref, dinv_ref, b_ref, w_ref, lo_ref, hi_ref):
    dinv = dinv_ref[...]
    h_lo = jnp.maximum(dinv * plo_ref[...] + b_ref[:, :DH], 0.0)
    h_hi = jnp.maximum(dinv * phi_ref[...] + b_ref[:, DH:], 0.0)
    m = (jnp.dot(h_lo, w_ref[:DH, :], preferred_element_type=jnp.float32)
         + jnp.dot(h_hi, w_ref[DH:, :], preferred_element_type=jnp.float32))
    xs = dinv * m
    lo_ref[...] = xs[:, :DH].astype(jnp.bfloat16)
    hi_ref[...] = xs[:, DH:].astype(jnp.bfloat16)


def _fin_body(plo_ref, phi_ref, dinv_ref, b_ref, wo_ref, bo_ref,
              h_ref, out_ref):
    dinv = dinv_ref[...]
    h_lo = dinv * plo_ref[...] + b_ref[:, :DH]
    h_hi = dinv * phi_ref[...] + b_ref[:, DH:]
    h_ref[:, :DH] = h_lo
    h_ref[:, DH:] = h_hi
    out_ref[...] = (
        jnp.dot(h_lo, wo_ref[:DH, :], preferred_element_type=jnp.float32)
        + jnp.dot(h_hi, wo_ref[DH:, :], preferred_element_type=jnp.float32)
        + bo_ref[...])


def _rows_spec(width):
    return pl.BlockSpec((BLK, width), lambda i: (i, 0))


def _full_spec(shape):
    return pl.BlockSpec(shape, lambda i: tuple(0 for _ in shape))


def _tc_c1(deg_parts, x_pad, W1p):
    return pl.pallas_call(
        _c1_body,
        grid=(GRID,),
        in_specs=[
            pl.BlockSpec((NC, BLK, 1), lambda i: (0, i, 0)),
            _rows_spec(D),
            _full_spec((D, D)),
        ],
        out_specs=[_rows_spec(1), _rows_spec(DH), _rows_spec(DH)],
        out_shape=[
            jax.ShapeDtypeStruct((NPAD, 1), jnp.float32),
            jax.ShapeDtypeStruct((NPAD, DH), jnp.bfloat16),
            jax.ShapeDtypeStruct((NPAD, DH), jnp.bfloat16),
        ],
    )(deg_parts, x_pad, W1p)


def _tc_mid(p_lo, p_hi, dinv, b, Wp):
    return pl.pallas_call(
        _mid_body,
        grid=(GRID,),
        in_specs=[
            _rows_spec(DH), _rows_spec(DH),
            _rows_spec(1),
            _full_spec((1, D)),
            _full_spec((D, D)),
        ],
        out_specs=[_rows_spec(DH), _rows_spec(DH)],
        out_shape=[
            jax.ShapeDtypeStruct((NPAD, DH), jnp.bfloat16),
            jax.ShapeDtypeStruct((NPAD, DH), jnp.bfloat16),
        ],
    )(p_lo, p_hi, dinv, b, Wp)


def _tc_fin(p_lo, p_hi, dinv, b, Wo, bo):
    return pl.pallas_call(
        _fin_body,
        grid=(GRID,),
        in_specs=[
            _rows_spec(DH), _rows_spec(DH),
            _rows_spec(1),
            _full_spec((1, D)),
            _full_spec((D, 1)),
            _full_spec((1, 1)),
        ],
        out_specs=[_rows_spec(D), _rows_spec(1)],
        out_shape=[
            jax.ShapeDtypeStruct((NPAD, D), jnp.float32),
            jax.ShapeDtypeStruct((NPAD, 1), jnp.float32),
        ],
    )(p_lo, p_hi, dinv, b, Wo, bo)


def kernel(x, edge_index, W1, b1, W2, b2, W3, b3, Wo, bo):
    src = edge_index[0].astype(jnp.int32)
    dst = edge_index[1].astype(jnp.int32)
    loop = jnp.arange(NPAD, dtype=jnp.int32)
    pad1 = jnp.full((EROWS_DEG * 128 - E_EDGES,), N_NODES, jnp.int32)
    pad2 = jnp.full(((EROWS - 2640) * 128,), N_NODES, jnp.int32)
    src = jnp.concatenate([src, pad1, loop, pad2])
    dst = jnp.concatenate([dst, pad1, loop, pad2])
    dstx = dst.reshape(EROWS, 128)
    edgx = jnp.stack([src.reshape(EBROWS, EB), dst.reshape(EBROWS, EB)],
                     axis=1)

    x_pad = jnp.zeros((NPAD, D), jnp.float32).at[:N_NODES].set(x)
    perm = _col_perm()
    W1p = W1[:, perm]
    W2p = W2[:, perm]
    W3p = W3[:, perm]

    deg_parts = _deg_kernel(dstx)
    deg_parts = deg_parts.reshape(NC, NPAD, 1)

    dinv, xs1_lo, xs1_hi = _tc_c1(deg_parts, x_pad, W1p)
    p1_lo, p1_hi = _agg_kernel(xs1_lo, xs1_hi, edgx)
    xs2_lo, xs2_hi = _tc_mid(p1_lo, p1_hi, dinv, b1.reshape(1, D), W2p)
    p2_lo, p2_hi = _agg_kernel(xs2_lo, xs2_hi, edgx)
    xs3_lo, xs3_hi = _tc_mid(p2_lo, p2_hi, dinv, b2.reshape(1, D), W3p)
    p3_lo, p3_hi = _agg_kernel(xs3_lo, xs3_hi, edgx)
    h, out = _tc_fin(p3_lo, p3_hi, dinv, b3.reshape(1, D),
                     Wo, bo.reshape(1, 1))
    return (out[:N_NODES], h[:N_NODES])


# speedup vs baseline: 1.0159x; 1.0159x over previous
import functools

import jax
import jax.numpy as jnp
import numpy as np
from jax import lax
from jax.experimental import pallas as pl
from jax.experimental.pallas import tpu as pltpu
from jax.experimental.pallas import tpu_sc as plsc

N_NODES = 10000
NPAD = 10240
D = 128
DH = 64
E_EDGES = 320000
EROWS_REAL = E_EDGES // 128
EROWS_DEG = 2560
EROWS = 2688
EPAD = EROWS * 128
NC = 2
NS = 16
KDEG = EROWS_DEG // (NC * NS)
EB = 128
EBROWS = EROWS * 128 // EB
KAGG = EBROWS // NS
RPT = NPAD // NS
BLK = 1024
GRID = NPAD // BLK

NBUF = 6
NRING = 8
STAG = 3


def _col_perm():
    perm = np.empty((D,), np.int64)
    for b0 in range(0, D, 32):
        perm[b0:b0 + 32:2] = b0 + np.arange(16)
        perm[b0 + 1:b0 + 32:2] = b0 + 16 + np.arange(16)
    return perm


def _sc_mesh():
    return plsc.VectorSubcoreMesh(core_axis_name="c", subcore_axis_name="s",
                                  num_cores=NC, num_subcores=NS)


def _deg_body(dstx_hbm, out_hbm, dst_v, ones_v, z_v, acc):
    cid = lax.axis_index("c")
    sid = lax.axis_index("s")
    wid = cid * NS + sid

    pltpu.sync_copy(dstx_hbm.at[pl.ds(wid * KDEG, KDEG)], dst_v)

    def fill(i, _):
        ones_v[pl.ds(i * 16, 16)] = jnp.full((16,), 1.0, jnp.float32)
        z_v[pl.ds(i * 16, 16)] = jnp.zeros((16,), jnp.float32)
        return 0

    lax.fori_loop(0, 8, fill, 0)

    def zcp(t, _):
        pltpu.sync_copy(z_v, acc.at[pl.ds(sid * RPT + t * 128, 128)])
        return 0

    lax.fori_loop(0, RPT // 128, zcp, 0)
    plsc.subcore_barrier()

    def body(j, _):
        pltpu.sync_copy(ones_v, acc.at[dst_v.at[j]], add=True)
        return 0

    lax.fori_loop(0, KDEG, body, 0)
    plsc.subcore_barrier()
    pltpu.sync_copy(
        acc.at[pl.ds(sid * RPT, RPT)],
        out_hbm.at[cid, pl.ds(sid * RPT, RPT)],
    )


@functools.cache
def _build_deg_kernel():
    return functools.partial(
        pl.kernel,
        out_type=jax.ShapeDtypeStruct((NC, NPAD), jnp.float32),
        mesh=_sc_mesh(),
        scratch_types=[
            pltpu.VMEM((KDEG, 128), jnp.int32),
            pltpu.VMEM((128,), jnp.float32),
            pltpu.VMEM((128,), jnp.float32),
            pltpu.VMEM_SHARED((NPAD,), jnp.float32),
        ],
    )(_deg_body)


def _deg_kernel(dstx):
    return _build_deg_kernel()(dstx)


def _agg_half(xs_hbm, edgx_hbm, out_hbm, sid, ribuf, gbufs, fbufs, isems,
              gsems, ssems, z_v, acc):
    base = sid * KAGG

    def zcp(t, _):
        pltpu.sync_copy(z_v, acc.at[pl.ds(sid * RPT + t * 64, 64)])
        return 0

    lax.fori_loop(0, RPT // 64, zcp, 0)
    plsc.subcore_barrier()

    def i_desc(j):
        return pltpu.make_async_copy(edgx_hbm.at[base + j],
                                     ribuf.at[lax.rem(j, NRING)],
                                     isems[0])

    def g_desc(j, b):
        return pltpu.make_async_copy(
            xs_hbm.at[ribuf.at[lax.rem(j, NRING), 0]], gbufs[b], gsems[b])

    def s_desc(j, b):
        return pltpu.make_async_copy(
            fbufs[b], acc.at[ribuf.at[lax.rem(j, NRING), 1]], ssems[b])

    def s_start(j, b):
        pltpu.async_copy(fbufs[b], acc.at[ribuf.at[lax.rem(j, NRING), 1]],
                         ssems[b], add=True)

    def convert(b):
        gbuf = gbufs[b]
        fbuf = fbufs[b]

        def conv_row(r, _):
            for g in range(2):
                ab = gbuf[r, pl.ds(g * 32, 32)]
                lo16, hi16 = plsc.unpack(
                    ab, format=plsc.PackFormat.INTERLEAVED,
                    preferred_element_type=jnp.float32)
                fbuf[r, pl.ds(g * 32, 16)] = lo16
                fbuf[r, pl.ds(g * 32 + 16, 16)] = hi16
            return 0

        lax.fori_loop(0, EB, conv_row, 0)

    for j0 in range(NRING):
        i_desc(j0).start()
    for b0 in range(STAG):
        i_desc(b0).wait()
        g_desc(b0, b0).start()

    def body(i, _):
        for b in range(NBUF):
            j = i * NBUF + b
            g_desc(j, b).wait()
            convert(b)
            s_start(j, b)
            j2 = j - STAG
            b2 = (b + STAG) % NBUF

            @pl.when(j2 + NBUF < KAGG)
            def _():
                i_desc(j2 + NBUF).wait()
                g_desc(j2 + NBUF, b2).start()

            @pl.when(j2 >= 0)
            def _():
                s_desc(j2, b2).wait()

            @pl.when(jnp.logical_and(j2 >= 0, j2 + NRING < KAGG))
            def _():
                i_desc(j2 + NRING).start()

        return 0

    lax.fori_loop(0, KAGG // NBUF, body, 0)
    for jt in range(KAGG - STAG, KAGG):
        s_desc(jt, jt % NBUF).wait()
    plsc.subcore_barrier()
    pltpu.sync_copy(
        acc.at[pl.ds(sid * RPT, RPT)],
        out_hbm.at[pl.ds(sid * RPT, RPT)],
    )


def _agg_body(xs_lo_hbm, xs_hi_hbm, edgx_hbm, out_lo_hbm,
              out_hi_hbm, ribuf, gbuf0, gbuf1, gbuf2, gbuf3, gbuf4, gbuf5,
              fbuf0, fbuf1, fbuf2, fbuf3, fbuf4, fbuf5, isem,
              gsem0, gsem1, gsem2, gsem3, gsem4, gsem5,
              ssem0, ssem1, ssem2, ssem3, ssem4, ssem5,
              z_v, acc):
    cid = lax.axis_index("c")
    sid = lax.axis_index("s")

    def zfill(i, _):
        for k in range(4):
            z_v[i, pl.ds(k * 16, 16)] = jnp.zeros((16,), jnp.float32)
        return 0

    lax.fori_loop(0, 64, zfill, 0)

    gbufs = (gbuf0, gbuf1, gbuf2, gbuf3, gbuf4, gbuf5)
    fbufs = (fbuf0, fbuf1, fbuf2, fbuf3, fbuf4, fbuf5)
    isems = (isem,)
    gsems = (gsem0, gsem1, gsem2, gsem3, gsem4, gsem5)
    ssems = (ssem0, ssem1, ssem2, ssem3, ssem4, ssem5)

    @pl.when(cid == 0)
    def _():
        _agg_half(xs_lo_hbm, edgx_hbm, out_lo_hbm, sid, ribuf, gbufs, fbufs,
                  isems, gsems, ssems, z_v, acc)

    @pl.when(cid == 1)
    def _():
        _agg_half(xs_hi_hbm, edgx_hbm, out_hi_hbm, sid, ribuf, gbufs, fbufs,
                  isems, gsems, ssems, z_v, acc)


@functools.cache
def _build_agg_kernel():
    return functools.partial(
        pl.kernel,
        out_type=[
            jax.ShapeDtypeStruct((NPAD, DH), jnp.float32),
            jax.ShapeDtypeStruct((NPAD, DH), jnp.float32),
        ],
        mesh=_sc_mesh(),
        scratch_types=(
            [pltpu.VMEM((NRING, 2, EB), jnp.int32)]
            + [pltpu.VMEM((EB, DH), jnp.bfloat16)] * NBUF
            + [pltpu.VMEM((EB, DH), jnp.float32)] * NBUF
            + [pltpu.SemaphoreType.DMA] * (1 + 2 * NBUF)
            + [
                pltpu.VMEM((64, DH), jnp.float32),
                pltpu.VMEM_SHARED((NPAD, DH), jnp.float32),
            ]
        ),
        compiler_params=pltpu.CompilerParams(use_tc_tiling_on_sc=False,
                                             needs_layout_passes=False),
    )(_agg_body)


def _agg_kernel(xs_lo, xs_hi, edgx):
    return _build_agg_kernel()(xs_lo, xs_hi, edgx)


def _row_iota(i):
    return lax.broadcasted_iota(jnp.int32, (BLK, 1), 0) + i * BLK


def _c1_body(degp_ref, x_ref, w_ref, dinv_ref, lo_ref, hi_ref):
    i = pl.program_id(0)
    deg = degp_ref[0] + degp_ref[1] + 1.0
    dinv = jnp.where(_row_iota(i) < N_NODES, lax.rsqrt(deg), 0.0)
    dinv_ref[...] = dinv
    xs = dinv * jnp.dot(x_ref[...], w_ref[...],
                        preferred_element_type=jnp.float32)
    lo_ref[...] = xs[:, :DH].astype(jnp.bfloat16)
    hi_ref[...] = xs[:, DH:].astype(jnp.bfloat16)


def _mid_body(plo_ref, phi_ref, dinv_ref, b_ref, w_ref, lo_ref, hi_ref):
    dinv = dinv_ref[...]
    h_lo = jnp.maximum(dinv * plo_ref[...] + b_ref[:, :DH], 0.0)
    h_hi = jnp.maximum(dinv * phi_ref[...] + b_ref[:, DH:], 0.0)
    m = (jnp.dot(h_lo, w_ref[:DH, :], preferred_element_type=jnp.float32)
         + jnp.dot(h_hi, w_ref[DH:, :], preferred_element_type=jnp.float32))
    xs = dinv * m
    lo_ref[...] = xs[:, :DH].astype(jnp.bfloat16)
    hi_ref[...] = xs[:, DH:].astype(jnp.bfloat16)


def _fin_body(plo_ref, phi_ref, dinv_ref, b_ref, wo_ref, bo_ref,
              h_ref, out_ref):
    dinv = dinv_ref[...]
    h_lo = dinv * plo_ref[...] + b_ref[:, :DH]
    h_hi = dinv * phi_ref[...] + b_ref[:, DH:]
    h_ref[:, :DH] = h_lo
    h_ref[:, DH:] = h_hi
    out_ref[...] = (
        jnp.dot(h_lo, wo_ref[:DH, :], preferred_element_type=jnp.float32)
        + jnp.dot(h_hi, wo_ref[DH:, :], preferred_element_type=jnp.float32)
        + bo_ref[...])


def _rows_spec(width):
    return pl.BlockSpec((BLK, width), lambda i: (i, 0))


def _full_spec(shape):
    return pl.BlockSpec(shape, lambda i: tuple(0 for _ in shape))


def _tc_c1(deg_parts, x_pad, W1p):
    return pl.pallas_call(
        _c1_body,
        grid=(GRID,),
        in_specs=[
            pl.BlockSpec((NC, BLK, 1), lambda i: (0, i, 0)),
            _rows_spec(D),
            _full_spec((D, D)),
        ],
        out_specs=[_rows_spec(1), _rows_spec(DH), _rows_spec(DH)],
        out_shape=[
            jax.ShapeDtypeStruct((NPAD, 1), jnp.float32),
            jax.ShapeDtypeStruct((NPAD, DH), jnp.bfloat16),
            jax.ShapeDtypeStruct((NPAD, DH), jnp.bfloat16),
        ],
    )(deg_parts, x_pad, W1p)


def _tc_mid(p_lo, p_hi, dinv, b, Wp):
    return pl.pallas_call(
        _mid_body,
        grid=(GRID,),
        in_specs=[
            _rows_spec(DH), _rows_spec(DH),
            _rows_spec(1),
            _full_spec((1, D)),
            _full_spec((D, D)),
        ],
        out_specs=[_rows_spec(DH), _rows_spec(DH)],
        out_shape=[
            jax.ShapeDtypeStruct((NPAD, DH), jnp.bfloat16),
            jax.ShapeDtypeStruct((NPAD, DH), jnp.bfloat16),
        ],
    )(p_lo, p_hi, dinv, b, Wp)


def _tc_fin(p_lo, p_hi, dinv, b, Wo, bo):
    return pl.pallas_call(
        _fin_body,
        grid=(GRID,),
        in_specs=[
            _rows_spec(DH), _rows_spec(DH),
            _rows_spec(1),
            _full_spec((1, D)),
            _full_spec((D, 1)),
            _full_spec((1, 1)),
        ],
        out_specs=[_rows_spec(D), _rows_spec(1)],
        out_shape=[
            jax.ShapeDtypeStruct((NPAD, D), jnp.float32),
            jax.ShapeDtypeStruct((NPAD, 1), jnp.float32),
        ],
    )(p_lo, p_hi, dinv, b, Wo, bo)


def kernel(x, edge_index, W1, b1, W2, b2, W3, b3, Wo, bo):
    src = edge_index[0].astype(jnp.int32)
    dst = edge_index[1].astype(jnp.int32)
    loop = jnp.arange(NPAD, dtype=jnp.int32)
    pad1 = jnp.full((EROWS_DEG * 128 - E_EDGES,), N_NODES, jnp.int32)
    pad2 = jnp.full(((EROWS - 2640) * 128,), N_NODES, jnp.int32)
    src = jnp.concatenate([src, pad1, loop, pad2])
    dst = jnp.concatenate([dst, pad1, loop, pad2])
    dstx = dst.reshape(EROWS, 128)
    edgx = jnp.stack([src.reshape(EBROWS, EB), dst.reshape(EBROWS, EB)],
                     axis=1)

    x_pad = jnp.zeros((NPAD, D), jnp.float32).at[:N_NODES].set(x)
    perm = _col_perm()
    W1p = W1[:, perm]
    W2p = W2[:, perm]
    W3p = W3[:, perm]

    deg_parts = _deg_kernel(dstx)
    deg_parts = deg_parts.reshape(NC, NPAD, 1)

    dinv, xs1_lo, xs1_hi = _tc_c1(deg_parts, x_pad, W1p)
    p1_lo, p1_hi = _agg_kernel(xs1_lo, xs1_hi, edgx)
    xs2_lo, xs2_hi = _tc_mid(p1_lo, p1_hi, dinv, b1.reshape(1, D), W2p)
    p2_lo, p2_hi = _agg_kernel(xs2_lo, xs2_hi, edgx)
    xs3_lo, xs3_hi = _tc_mid(p2_lo, p2_hi, dinv, b2.reshape(1, D), W3p)
    p3_lo, p3_hi = _agg_kernel(xs3_lo, xs3_hi, edgx)
    h, out = _tc_fin(p3_lo, p3_hi, dinv, b3.reshape(1, D),
                     Wo, bo.reshape(1, 1))
    return (out[:N_NODES], h[:N_NODES])
